# baseline (speedup 1.0000x reference)
import jax
import jax.numpy as jnp
from jax.experimental import pallas as pl


def _norm(x, keepdims=False):
    return jnp.sqrt(jnp.sum(x * x, axis=-1, keepdims=keepdims) + 1e-12)


def _feat(pos, faces):
    v0 = pos[faces[:, 0]]
    v1 = pos[faces[:, 1]]
    v2 = pos[faces[:, 2]]
    e0 = v1 - v0
    e1 = v2 - v1
    e2 = v0 - v2
    n = jnp.cross(e0, e1)
    nn = _norm(n, keepdims=True)
    nrm = n / (nn + 1e-8)
    area = 0.5 * nn
    lens = jnp.stack([_norm(e0), _norm(e1), _norm(e2)], axis=1)
    return jnp.concatenate([nrm, area, lens], axis=1)


def _node_mean(faces, x, n_nodes):
    idx = faces.reshape(-1)
    vals = jnp.repeat(x, 3, axis=0)
    s = jnp.zeros((n_nodes, x.shape[1]), x.dtype).at[idx].add(vals)
    c = jnp.zeros((n_nodes,), x.dtype).at[idx].add(1.0)
    return s / jnp.maximum(c, 1.0)[:, None]


def _face_conv(faces, x, W, b, n_nodes):
    nm = _node_mean(faces, x, n_nodes)
    neigh = (nm[faces[:, 0]] + nm[faces[:, 1]] + nm[faces[:, 2]]) / 3.0
    return jnp.concatenate([x, neigh], axis=1) @ W + b


def _face2node(pos, faces, ff, Wp, bp, Wf, bf, n_nodes):
    nf = _node_mean(faces, ff, n_nodes)
    new_pos = pos + nf @ Wp + bp
    if Wf is None:
        return nf, new_pos, None
    face_nf = (nf[faces[:, 0]] + nf[faces[:, 1]] + nf[faces[:, 2]]) / 3.0
    return nf, new_pos, face_nf @ Wf + bf


def _identity_kernel(x_ref, o_ref):
    o_ref[...] = x_ref[...]


def kernel(pos, faces, W1, b1, W1a, b1a, Wp1, bp1, Wf1, bf1, W2, b2, W2a, b2a, Wp2, bp2, Wf2, bf2, W3, b3, W3a, b3a, Wp3, bp3):
    n = pos.shape[0]
    ff = jax.nn.relu(_face_conv(faces, _feat(pos, faces), W1, b1, n))
    ff = jax.nn.relu(_face_conv(faces, ff, W1a, b1a, n))
    _, pos, ff = _face2node(pos, faces, ff, Wp1, bp1, Wf1, bf1, n)
    ff = jax.nn.relu(_face_conv(faces, ff, W2, b2, n))
    ff = jax.nn.relu(_face_conv(faces, ff, W2a, b2a, n))
    _, pos, ff = _face2node(pos, faces, ff, Wp2, bp2, Wf2, bf2, n)
    ff = jax.nn.relu(_face_conv(faces, ff, W3, b3, n))
    ff = jax.nn.relu(_face_conv(faces, ff, W3a, b3a, n))
    _, pos, _ = _face2node(pos, faces, ff, Wp3, bp3, None, None, n)
    pos = pl.pallas_call(
        _identity_kernel,
        grid=(20,),
        in_specs=[pl.BlockSpec((5000, 3), lambda i: (i, 0))],
        out_specs=pl.BlockSpec((5000, 3), lambda i: (i, 0)),
        out_shape=jax.ShapeDtypeStruct(pos.shape, pos.dtype),
    )(pos)
    return pos


# speedup vs baseline: 1.5541x; 1.5541x over previous
import functools

import jax
import jax.numpy as jnp
from jax import lax
from jax.experimental import pallas as pl
from jax.experimental.pallas import tpu as pltpu, tpu_sc as plsc

N = 100000
F = 200000
NP = 102400
FP = 212992
HALF = FP // 2
CHUNK = 6656
SW = 32
SKW = CHUNK // SW
WB = 32
NZ = NP // 16
NC = 2
BN = 512

_mesh = lambda: plsc.VectorSubcoreMesh(
    core_axis_name="c", subcore_axis_name="s", num_cores=2, num_subcores=16)


def _make_scatter(ngroups, with_x):
    scratch = [
        pltpu.VMEM_SHARED((NP, 16), jnp.float32),
        pltpu.VMEM((SW,), jnp.int32),
        pltpu.VMEM((SW,), jnp.int32),
        pltpu.VMEM((SW,), jnp.int32),
        pltpu.VMEM((WB,), jnp.int32),
        pltpu.VMEM((SW, 16), jnp.float32),
        pltpu.VMEM((WB, 128), jnp.float32),
        pltpu.VMEM((WB, 16), jnp.float32),
    ]
    if with_x:
        scratch.append(pltpu.VMEM((SW, 128), jnp.float32))

    def body(*refs):
        if with_x:
            (x_hbm, f0_hbm, f1_hbm, f2_hbm, out_hbm,
             acc, i0, i1, i2, seq, vals, wbuf, abuf, xbuf) = refs
        else:
            (f0_hbm, f1_hbm, f2_hbm, out_hbm,
             acc, i0, i1, i2, seq, vals, wbuf, abuf) = refs
        c = lax.axis_index("c")
        s = lax.axis_index("s")
        z16 = jnp.zeros((16,), jnp.float32)
        iota16 = lax.iota(jnp.int32, 16)
        base = c * HALF + s * CHUNK
        nzb = s * NZ

        def zwb(j, _):
            for lk in range(8):
                wbuf[j, pl.ds(lk * 16, 16)] = z16
            return 0
        lax.fori_loop(0, WB, zwb, 0)

        if not with_x:
            one16 = jnp.full((16,), 1.0, jnp.float32)

            def fv(j, _):
                vals[j, :] = one16
                return 0
            lax.fori_loop(0, SW, fv, 0)

        for g in range(ngroups):
            def zab(j, _):
                abuf[j, :] = z16
                return 0
            lax.fori_loop(0, WB, zab, 0)

            def zacc(j, _):
                for k in range(2):
                    seq[pl.ds(k * 16, 16)] = iota16 + (nzb + j * WB + k * 16)
                pltpu.sync_copy(abuf, acc.at[seq])
                return 0
            lax.fori_loop(0, NZ // WB, zacc, 0)
            plsc.subcore_barrier()

            def sbody(j, _):
                off = base + j * SW
                pltpu.sync_copy(f0_hbm.at[pl.ds(off, SW)], i0)
                pltpu.sync_copy(f1_hbm.at[pl.ds(off, SW)], i1)
                pltpu.sync_copy(f2_hbm.at[pl.ds(off, SW)], i2)
                if with_x:
                    pltpu.sync_copy(x_hbm.at[pl.ds(off, SW)], xbuf)

                    def ext(j2, _):
                        vals[j2, :] = xbuf[j2, pl.ds(g * 16, 16)]
                        return 0
                    lax.fori_loop(0, SW, ext, 0)
                pltpu.sync_copy(vals, acc.at[i0], add=True)
                pltpu.sync_copy(vals, acc.at[i1], add=True)
                pltpu.sync_copy(vals, acc.at[i2], add=True)
                return 0
            for t in range(16):
                @pl.when(s == t)
                def _():
                    lax.fori_loop(0, SKW, sbody, 0)
                plsc.subcore_barrier()
            plsc.subcore_barrier()

            def rb(j, _):
                for k in range(2):
                    seq[pl.ds(k * 16, 16)] = iota16 + (nzb + j * WB + k * 16)
                pltpu.sync_copy(acc.at[seq], abuf)
                row0 = c * NP + nzb + j * WB
                if g > 0:
                    pltpu.sync_copy(out_hbm.at[pl.ds(row0, WB)], wbuf)

                def mv(j2, _):
                    wbuf[j2, pl.ds(g * 16, 16)] = abuf[j2, :]
                    return 0
                lax.fori_loop(0, WB, mv, 0)
                pltpu.sync_copy(wbuf, out_hbm.at[pl.ds(row0, WB)])
                return 0
            lax.fori_loop(0, NZ // WB, rb, 0)
            plsc.subcore_barrier()

    out_type = jax.ShapeDtypeStruct((2 * NP, 128), jnp.float32)
    return functools.partial(
        pl.kernel, body, out_type=out_type, mesh=_mesh(),
        scratch_types=scratch)()


def _make_gather():
    GW = 512
    KW = CHUNK // GW

    @functools.partial(
        pl.kernel,
        out_type=jax.ShapeDtypeStruct((FP, 128), jnp.float32),
        mesh=_mesh(),
        scratch_types=[
            pltpu.VMEM((GW,), jnp.int32),
            pltpu.VMEM((GW,), jnp.int32),
            pltpu.VMEM((GW,), jnp.int32),
            pltpu.VMEM((GW, 128), jnp.float32),
            pltpu.SemaphoreType.DMA,
        ],
    )
    def k(y_hbm, f0_hbm, f1_hbm, f2_hbm, out_hbm, i0, i1, i2, rows, sem):
        c = lax.axis_index("c")
        s = lax.axis_index("s")
        w = s * NC + c
        base = w * (KW * GW)

        def body(j, _):
            off = base + j * GW
            pltpu.sync_copy(f0_hbm.at[pl.ds(off, GW)], i0)
            pltpu.sync_copy(f1_hbm.at[pl.ds(off, GW)], i1)
            pltpu.sync_copy(f2_hbm.at[pl.ds(off, GW)], i2)
            pltpu.async_copy(y_hbm.at[i0], rows, sem).wait()
            pltpu.async_copy(y_hbm.at[i1], rows, sem, add=True).wait()
            pltpu.async_copy(y_hbm.at[i2], rows, sem, add=True).wait()
            pltpu.sync_copy(rows, out_hbm.at[pl.ds(off, GW)])
            return 0
        lax.fori_loop(0, KW, body, 0)

    return k


def _make_featgather():
    GW = 128
    KW = CHUNK // GW

    @functools.partial(
        pl.kernel,
        out_type=jax.ShapeDtypeStruct((FP, 128), jnp.float32),
        mesh=_mesh(),
        scratch_types=[
            pltpu.VMEM((GW,), jnp.int32),
            pltpu.VMEM((GW,), jnp.int32),
            pltpu.VMEM((GW,), jnp.int32),
            pltpu.VMEM((GW, 128), jnp.float32),
            pltpu.VMEM((GW, 128), jnp.float32),
            pltpu.VMEM((GW, 128), jnp.float32),
            pltpu.VMEM((GW, 128), jnp.float32),
            pltpu.SemaphoreType.DMA,
        ],
    )
    def k(p_hbm, f0_hbm, f1_hbm, f2_hbm, out_hbm,
          i0, i1, i2, r0, r1, r2, vp, sem):
        c = lax.axis_index("c")
        s = lax.axis_index("s")
        w = s * NC + c
        base = w * CHUNK

        def body(j, _):
            off = base + j * GW
            pltpu.sync_copy(f0_hbm.at[pl.ds(off, GW)], i0)
            pltpu.sync_copy(f1_hbm.at[pl.ds(off, GW)], i1)
            pltpu.sync_copy(f2_hbm.at[pl.ds(off, GW)], i2)
            pltpu.async_copy(p_hbm.at[i0], r0, sem).wait()
            pltpu.async_copy(p_hbm.at[i1], r1, sem).wait()
            pltpu.async_copy(p_hbm.at[i2], r2, sem).wait()

            def mv(j2, _):
                vp[j2, pl.ds(0, 16)] = r0[j2, pl.ds(0, 16)]
                vp[j2, pl.ds(16, 16)] = r1[j2, pl.ds(0, 16)]
                vp[j2, pl.ds(32, 16)] = r2[j2, pl.ds(0, 16)]
                return 0
            lax.fori_loop(0, GW, mv, 0)
            pltpu.sync_copy(vp, out_hbm.at[pl.ds(off, GW)])
            return 0
        lax.fori_loop(0, KW, body, 0)

    return k


def _blk(i):
    return (i, 0)


def _blk2(off):
    return lambda i: (i + off, 0)


def _tc_inv():
    def body(pa, pb, inv_o, ind_o):
        cnt = (pa[...] + pb[...])[:, :1]
        inv = 1.0 / jnp.maximum(cnt, 1.0)
        ind = jnp.minimum(cnt, 1.0)
        inv_o[...] = jnp.broadcast_to(inv, (BN, 128))
        ind_o[...] = jnp.broadcast_to(ind, (BN, 128))

    return pl.pallas_call(
        body,
        grid=(NP // BN,),
        in_specs=[pl.BlockSpec((BN, 128), _blk),
                  pl.BlockSpec((BN, 128), _blk2(NP // BN))],
        out_specs=[pl.BlockSpec((BN, 128), _blk),
                   pl.BlockSpec((BN, 128), _blk)],
        out_shape=[jax.ShapeDtypeStruct((NP, 128), jnp.float32),
                   jax.ShapeDtypeStruct((NP, 128), jnp.float32)],
    )


def _tc_nodemm():
    def body(pa, pb, inv, ind, w_ref, nb_ref, o_ref):
        nm = (pa[...] + pb[...]) * inv[...]
        y = jnp.dot(nm, w_ref[...], preferred_element_type=jnp.float32)
        o_ref[...] = y + ind[...][:, :1] * nb_ref[...]

    return pl.pallas_call(
        body,
        grid=(NP // BN,),
        in_specs=[pl.BlockSpec((BN, 128), _blk),
                  pl.BlockSpec((BN, 128), _blk2(NP // BN)),
                  pl.BlockSpec((BN, 128), _blk),
                  pl.BlockSpec((BN, 128), _blk),
                  pl.BlockSpec((128, 128), lambda i: (0, 0)),
                  pl.BlockSpec((1, 128), lambda i: (0, 0))],
        out_specs=pl.BlockSpec((BN, 128), _blk),
        out_shape=jax.ShapeDtypeStruct((NP, 128), jnp.float32),
    )


def _tc_facemm():
    def body(x_ref, g_ref, w_ref, b_ref, o_ref):
        y = jnp.dot(x_ref[...], w_ref[...], preferred_element_type=jnp.float32)
        o_ref[...] = jnp.maximum(y + g_ref[...] + b_ref[...], 0.0)

    return pl.pallas_call(
        body,
        grid=(FP // BN,),
        in_specs=[pl.BlockSpec((BN, 128), _blk),
                  pl.BlockSpec((BN, 128), _blk),
                  pl.BlockSpec((128, 128), lambda i: (0, 0)),
                  pl.BlockSpec((1, 128), lambda i: (0, 0))],
        out_specs=pl.BlockSpec((BN, 128), _blk),
        out_shape=jax.ShapeDtypeStruct((FP, 128), jnp.float32),
    )


def _tc_geom():
    def body(vp_ref, o_ref):
        b = vp_ref[...]
        v0 = b[:, 0:3]
        v1 = b[:, 16:19]
        v2 = b[:, 32:35]
        e0 = v1 - v0
        e1 = v2 - v1
        e2 = v0 - v2
        n0 = e0[:, 1:2] * e1[:, 2:3] - e0[:, 2:3] * e1[:, 1:2]
        n1 = e0[:, 2:3] * e1[:, 0:1] - e0[:, 0:1] * e1[:, 2:3]
        n2 = e0[:, 0:1] * e1[:, 1:2] - e0[:, 1:2] * e1[:, 0:1]
        nn = jnp.sqrt(n0 * n0 + n1 * n1 + n2 * n2 + 1e-12)
        inv_n = 1.0 / (nn + 1e-8)
        area = 0.5 * nn

        def ln(e):
            return jnp.sqrt(jnp.sum(e * e, axis=1, keepdims=True) + 1e-12)

        feats = jnp.concatenate(
            [n0 * inv_n, n1 * inv_n, n2 * inv_n, area, ln(e0), ln(e1), ln(e2),
             jnp.zeros((BN, 121), jnp.float32)], axis=1)
        o_ref[...] = feats

    return pl.pallas_call(
        body,
        grid=(FP // BN,),
        in_specs=[pl.BlockSpec((BN, 128), _blk)],
        out_specs=pl.BlockSpec((BN, 128), _blk),
        out_shape=jax.ShapeDtypeStruct((FP, 128), jnp.float32),
    )


def _tc_final():
    def body(p1a, p1b, p2a, p2b, p3a, p3b, inv, w1, w2, w3, bp, pos, o_ref):
        iv = inv[...]
        y = jnp.dot((p1a[...] + p1b[...]) * iv, w1[...],
                    preferred_element_type=jnp.float32)
        y += jnp.dot((p2a[...] + p2b[...]) * iv, w2[...],
                     preferred_element_type=jnp.float32)
        y += jnp.dot((p3a[...] + p3b[...]) * iv, w3[...],
                     preferred_element_type=jnp.float32)
        o_ref[...] = pos[...] + y[:, :3] + bp[...][:, :3]

    np_off = NP // BN
    return pl.pallas_call(
        body,
        grid=(NP // BN,),
        in_specs=[pl.BlockSpec((BN, 128), _blk),
                  pl.BlockSpec((BN, 128), _blk2(np_off)),
                  pl.BlockSpec((BN, 128), _blk),
                  pl.BlockSpec((BN, 128), _blk2(np_off)),
                  pl.BlockSpec((BN, 128), _blk),
                  pl.BlockSpec((BN, 128), _blk2(np_off)),
                  pl.BlockSpec((BN, 128), _blk),
                  pl.BlockSpec((128, 128), lambda i: (0, 0)),
                  pl.BlockSpec((128, 128), lambda i: (0, 0)),
                  pl.BlockSpec((128, 128), lambda i: (0, 0)),
                  pl.BlockSpec((1, 128), lambda i: (0, 0)),
                  pl.BlockSpec((BN, 3), _blk)],
        out_specs=pl.BlockSpec((BN, 3), _blk),
        out_shape=jax.ShapeDtypeStruct((NP, 3), jnp.float32),
    )


_scatter64 = None
_scatter16 = None
_counts = None
_gather = None
_featgather = None


def _kernels():
    global _scatter64, _scatter16, _counts, _gather, _featgather
    if _scatter64 is None:
        _scatter64 = _make_scatter(4, True)
        _scatter16 = _make_scatter(1, True)
        _counts = _make_scatter(1, False)
        _gather = _make_gather()
        _featgather = _make_featgather()
    return _scatter64, _scatter16, _counts, _gather, _featgather


def _padw(w):
    k, m = w.shape
    return jnp.zeros((128, 128), jnp.float32).at[:k, :m].set(w)


def _row(b):
    return jnp.zeros((1, 128), jnp.float32).at[0, :b.shape[0]].set(b)


def kernel(pos, faces, W1, b1, W1a, b1a, Wp1, bp1, Wf1, bf1, W2, b2, W2a, b2a,
           Wp2, bp2, Wf2, bf2, W3, b3, W3a, b3a, Wp3, bp3):
    sc64, sc16, counts_k, gather_k, featgather_k = _kernels()
    f32 = jnp.float32

    pad_n = FP - F
    trash = (100352 + (jnp.arange(pad_n, dtype=jnp.int32) % 2048))
    f0 = jnp.concatenate([faces[:, 0], trash])
    f1 = jnp.concatenate([faces[:, 1], trash])
    f2 = jnp.concatenate([faces[:, 2], trash])
    pos_pad = jnp.zeros((NP, 128), f32).at[:N, :3].set(pos)

    Wt1, Wb1 = _padw(W1[:7]), _padw(W1[7:] / 3.0)
    Wt1a, Wb1a = _padw(W1a[:64]), _padw(W1a[64:] / 3.0)
    Wt2, Wb2 = _padw(W2[:64]), _padw(W2[64:] / 3.0)
    Wt2a, Wb2a = _padw(W2a[:64]), _padw(W2a[64:] / 3.0)
    Wt3, Wb3 = _padw(W3[:64]), _padw(W3[64:] / 3.0)
    Wt3a, Wb3a = _padw(W3a[:64]), _padw(W3a[64:] / 3.0)
    Wfp1, Wfp2 = _padw(Wf1 / 3.0), _padw(Wf2 / 3.0)
    Wpp1, Wpp2, Wpp3 = _padw(Wp1), _padw(Wp2), _padw(Wp3)
    zrow = jnp.zeros((1, 128), f32)
    beff1, beff1a = _row(b1), _row(b1a)
    beff2 = _row(b2 + bf1 @ W2[:64])
    nbias2 = _row(bf1 @ (W2[64:] / 3.0))
    beff2a = _row(b2a)
    beff3 = _row(b3 + bf2 @ W3[:64])
    nbias3 = _row(bf2 @ (W3[64:] / 3.0))
    beff3a = _row(b3a)
    bpsum = _row(bp1 + bp2 + bp3)

    inv_k, nodemm, facemm, geom, final_k = (
        _tc_inv(), _tc_nodemm(), _tc_facemm(), _tc_geom(), _tc_final())

    def xla_scatter(x):
        s = jnp.zeros((NP, 64), f32)
        h = x[:, :64]
        s = s.at[f0].add(h).at[f1].add(h).at[f2].add(h)
        return jnp.zeros((2 * NP, 128), f32).at[:NP, :64].set(s)

    cnt = jnp.zeros((NP,), f32).at[f0].add(1.0).at[f1].add(1.0).at[f2].add(1.0)
    cpl = jnp.zeros((2 * NP, 128), f32).at[:NP, 0].set(cnt)
    inv, ind = inv_k(cpl, cpl)

    vpack = featgather_k(pos_pad, f0, f1, f2)
    x = geom(vpack)

    def face_conv(x, Wt, Wb, beff, nbias, wide):
        spl = xla_scatter(x)
        y = nodemm(spl, spl, inv, ind, Wb, nbias)
        g = gather_k(y, f0, f1, f2)
        return facemm(x, g, Wt, beff)

    x = face_conv(x, Wt1, Wb1, beff1, zrow, False)
    x = face_conv(x, Wt1a, Wb1a, beff1a, zrow, True)
    spl1 = xla_scatter(x)
    u = nodemm(spl1, spl1, inv, ind, Wfp1, zrow)
    x = gather_k(u, f0, f1, f2)
    x = face_conv(x, Wt2, Wb2, beff2, nbias2, True)
    x = face_conv(x, Wt2a, Wb2a, beff2a, zrow, True)
    spl2 = xla_scatter(x)
    u = nodemm(spl2, spl2, inv, ind, Wfp2, zrow)
    x = gather_k(u, f0, f1, f2)
    x = face_conv(x, Wt3, Wb3, beff3, nbias3, True)
    x = face_conv(x, Wt3a, Wb3a, beff3a, zrow, True)
    spl3 = xla_scatter(x)

    pos_out = final_k(spl1, spl1, spl2, spl2, spl3, spl3, inv,
                      Wpp1, Wpp2, Wpp3, bpsum,
                      jnp.zeros((NP, 3), f32).at[:N].set(pos))
    return pos_out[:N]


# speedup vs baseline: 1.5943x; 1.0259x over previous
import functools

import jax
import jax.numpy as jnp
from jax import lax
from jax.experimental import pallas as pl
from jax.experimental.pallas import tpu as pltpu, tpu_sc as plsc

N = 100000
F = 200000
NP = 102400
FP = 212992
HALF = FP // 2
CHUNK = 6656
SW = 32
SKW = CHUNK // SW
WB = 32
NZ = NP // 16
NC = 2
BN = 512

_mesh = lambda: plsc.VectorSubcoreMesh(
    core_axis_name="c", subcore_axis_name="s", num_cores=2, num_subcores=16)


def _make_scatter(ngroups, with_x):
    scratch = [
        pltpu.VMEM_SHARED((NP, 16), jnp.float32),
        pltpu.VMEM((SW,), jnp.int32),
        pltpu.VMEM((SW,), jnp.int32),
        pltpu.VMEM((SW,), jnp.int32),
        pltpu.VMEM((WB,), jnp.int32),
        pltpu.VMEM((SW, 16), jnp.float32),
        pltpu.VMEM((WB, 128), jnp.float32),
        pltpu.VMEM((WB, 16), jnp.float32),
    ]
    if with_x:
        scratch.append(pltpu.VMEM((SW, 128), jnp.float32))

    def body(*refs):
        if with_x:
            (x_hbm, f0_hbm, f1_hbm, f2_hbm, out_hbm,
             acc, i0, i1, i2, seq, vals, wbuf, abuf, xbuf) = refs
        else:
            (f0_hbm, f1_hbm, f2_hbm, out_hbm,
             acc, i0, i1, i2, seq, vals, wbuf, abuf) = refs
        c = lax.axis_index("c")
        s = lax.axis_index("s")
        z16 = jnp.zeros((16,), jnp.float32)
        iota16 = lax.iota(jnp.int32, 16)
        base = c * HALF + s * CHUNK
        nzb = s * NZ

        def zwb(j, _):
            for lk in range(8):
                wbuf[j, pl.ds(lk * 16, 16)] = z16
            return 0
        lax.fori_loop(0, WB, zwb, 0)

        if not with_x:
            one16 = jnp.full((16,), 1.0, jnp.float32)

            def fv(j, _):
                vals[j, :] = one16
                return 0
            lax.fori_loop(0, SW, fv, 0)

        for g in range(ngroups):
            def zab(j, _):
                abuf[j, :] = z16
                return 0
            lax.fori_loop(0, WB, zab, 0)

            def zacc(j, _):
                for k in range(2):
                    seq[pl.ds(k * 16, 16)] = iota16 + (nzb + j * WB + k * 16)
                pltpu.sync_copy(abuf, acc.at[seq])
                return 0
            lax.fori_loop(0, NZ // WB, zacc, 0)
            plsc.subcore_barrier()

            def sbody(j, _):
                off = base + j * SW
                pltpu.sync_copy(f0_hbm.at[pl.ds(off, SW)], i0)
                pltpu.sync_copy(f1_hbm.at[pl.ds(off, SW)], i1)
                pltpu.sync_copy(f2_hbm.at[pl.ds(off, SW)], i2)
                if with_x:
                    pltpu.sync_copy(x_hbm.at[pl.ds(off, SW)], xbuf)

                    def ext(j2, _):
                        vals[j2, :] = xbuf[j2, pl.ds(g * 16, 16)]
                        return 0
                    lax.fori_loop(0, SW, ext, 0)
                pltpu.sync_copy(vals, acc.at[i0], add=True)
                pltpu.sync_copy(vals, acc.at[i1], add=True)
                pltpu.sync_copy(vals, acc.at[i2], add=True)
                return 0
            for t in range(16):
                @pl.when(s == t)
                def _():
                    lax.fori_loop(0, SKW, sbody, 0)
                plsc.subcore_barrier()
            plsc.subcore_barrier()

            def rb(j, _):
                for k in range(2):
                    seq[pl.ds(k * 16, 16)] = iota16 + (nzb + j * WB + k * 16)
                pltpu.sync_copy(acc.at[seq], abuf)
                row0 = c * NP + nzb + j * WB
                if g > 0:
                    pltpu.sync_copy(out_hbm.at[pl.ds(row0, WB)], wbuf)

                def mv(j2, _):
                    wbuf[j2, pl.ds(g * 16, 16)] = abuf[j2, :]
                    return 0
                lax.fori_loop(0, WB, mv, 0)
                pltpu.sync_copy(wbuf, out_hbm.at[pl.ds(row0, WB)])
                return 0
            lax.fori_loop(0, NZ // WB, rb, 0)
            plsc.subcore_barrier()

    out_type = jax.ShapeDtypeStruct((2 * NP, 128), jnp.float32)
    return functools.partial(
        pl.kernel, body, out_type=out_type, mesh=_mesh(),
        scratch_types=scratch)()


def _make_gather():
    GW = 512
    KW = CHUNK // GW

    @functools.partial(
        pl.kernel,
        out_type=jax.ShapeDtypeStruct((FP, 128), jnp.float32),
        mesh=_mesh(),
        scratch_types=[
            pltpu.VMEM((GW,), jnp.int32),
            pltpu.VMEM((GW,), jnp.int32),
            pltpu.VMEM((GW,), jnp.int32),
            pltpu.VMEM((GW, 128), jnp.float32),
            pltpu.SemaphoreType.DMA,
        ],
    )
    def k(y_hbm, f0_hbm, f1_hbm, f2_hbm, out_hbm, i0, i1, i2, rows, sem):
        c = lax.axis_index("c")
        s = lax.axis_index("s")
        w = s * NC + c
        base = w * (KW * GW)

        def body(j, _):
            off = base + j * GW
            pltpu.sync_copy(f0_hbm.at[pl.ds(off, GW)], i0)
            pltpu.sync_copy(f1_hbm.at[pl.ds(off, GW)], i1)
            pltpu.sync_copy(f2_hbm.at[pl.ds(off, GW)], i2)
            pltpu.async_copy(y_hbm.at[i0], rows, sem).wait()
            pltpu.async_copy(y_hbm.at[i1], rows, sem, add=True).wait()
            pltpu.async_copy(y_hbm.at[i2], rows, sem, add=True).wait()
            pltpu.sync_copy(rows, out_hbm.at[pl.ds(off, GW)])
            return 0
        lax.fori_loop(0, KW, body, 0)

    return k


def _make_featgather():
    GW = 128
    KW = CHUNK // GW

    @functools.partial(
        pl.kernel,
        out_type=jax.ShapeDtypeStruct((FP, 128), jnp.float32),
        mesh=_mesh(),
        scratch_types=[
            pltpu.VMEM((GW,), jnp.int32),
            pltpu.VMEM((GW,), jnp.int32),
            pltpu.VMEM((GW,), jnp.int32),
            pltpu.VMEM((GW, 128), jnp.float32),
            pltpu.VMEM((GW, 128), jnp.float32),
            pltpu.VMEM((GW, 128), jnp.float32),
            pltpu.VMEM((GW, 128), jnp.float32),
            pltpu.SemaphoreType.DMA,
        ],
    )
    def k(p_hbm, f0_hbm, f1_hbm, f2_hbm, out_hbm,
          i0, i1, i2, r0, r1, r2, vp, sem):
        c = lax.axis_index("c")
        s = lax.axis_index("s")
        w = s * NC + c
        base = w * CHUNK

        def body(j, _):
            off = base + j * GW
            pltpu.sync_copy(f0_hbm.at[pl.ds(off, GW)], i0)
            pltpu.sync_copy(f1_hbm.at[pl.ds(off, GW)], i1)
            pltpu.sync_copy(f2_hbm.at[pl.ds(off, GW)], i2)
            pltpu.async_copy(p_hbm.at[i0], r0, sem).wait()
            pltpu.async_copy(p_hbm.at[i1], r1, sem).wait()
            pltpu.async_copy(p_hbm.at[i2], r2, sem).wait()

            def mv(j2, _):
                vp[j2, pl.ds(0, 16)] = r0[j2, pl.ds(0, 16)]
                vp[j2, pl.ds(16, 16)] = r1[j2, pl.ds(0, 16)]
                vp[j2, pl.ds(32, 16)] = r2[j2, pl.ds(0, 16)]
                return 0
            lax.fori_loop(0, GW, mv, 0)
            pltpu.sync_copy(vp, out_hbm.at[pl.ds(off, GW)])
            return 0
        lax.fori_loop(0, KW, body, 0)

    return k


def _blk(i):
    return (i, 0)


def _blk2(off):
    return lambda i: (i + off, 0)


def _tc_inv():
    def body(cnt_ref, inv_o, ind_o):
        cnt = cnt_ref[...]
        inv = 1.0 / jnp.maximum(cnt, 1.0)
        ind = jnp.minimum(cnt, 1.0)
        inv_o[...] = jnp.broadcast_to(inv, (BN, 64))
        ind_o[...] = jnp.broadcast_to(ind, (BN, 64))

    return pl.pallas_call(
        body,
        grid=(NP // BN,),
        in_specs=[pl.BlockSpec((BN, 1), _blk)],
        out_specs=[pl.BlockSpec((BN, 64), _blk),
                   pl.BlockSpec((BN, 64), _blk)],
        out_shape=[jax.ShapeDtypeStruct((NP, 64), jnp.float32),
                   jax.ShapeDtypeStruct((NP, 64), jnp.float32)],
    )


def _tc_nodemm():
    def body(s_ref, inv, ind, w_ref, nb_ref, o_ref):
        nm = s_ref[...] * inv[...]
        y = jnp.dot(nm, w_ref[...], preferred_element_type=jnp.float32)
        o_ref[...] = y + ind[...][:, :1] * nb_ref[...]

    return pl.pallas_call(
        body,
        grid=(NP // BN,),
        in_specs=[pl.BlockSpec((BN, 64), _blk),
                  pl.BlockSpec((BN, 64), _blk),
                  pl.BlockSpec((BN, 64), _blk),
                  pl.BlockSpec((64, 128), lambda i: (0, 0)),
                  pl.BlockSpec((1, 128), lambda i: (0, 0))],
        out_specs=pl.BlockSpec((BN, 128), _blk),
        out_shape=jax.ShapeDtypeStruct((NP, 128), jnp.float32),
    )


def _tc_facemm():
    def body(x_ref, g_ref, w_ref, b_ref, o_ref):
        y = jnp.dot(x_ref[...], w_ref[...], preferred_element_type=jnp.float32)
        o_ref[...] = jnp.maximum(y + g_ref[...] + b_ref[...], 0.0)

    return pl.pallas_call(
        body,
        grid=(FP // BN,),
        in_specs=[pl.BlockSpec((BN, 128), _blk),
                  pl.BlockSpec((BN, 128), _blk),
                  pl.BlockSpec((128, 128), lambda i: (0, 0)),
                  pl.BlockSpec((1, 128), lambda i: (0, 0))],
        out_specs=pl.BlockSpec((BN, 128), _blk),
        out_shape=jax.ShapeDtypeStruct((FP, 128), jnp.float32),
    )


def _tc_geom():
    def body(vp_ref, o_ref):
        b = vp_ref[...]
        v0 = b[:, 0:3]
        v1 = b[:, 16:19]
        v2 = b[:, 32:35]
        e0 = v1 - v0
        e1 = v2 - v1
        e2 = v0 - v2
        n0 = e0[:, 1:2] * e1[:, 2:3] - e0[:, 2:3] * e1[:, 1:2]
        n1 = e0[:, 2:3] * e1[:, 0:1] - e0[:, 0:1] * e1[:, 2:3]
        n2 = e0[:, 0:1] * e1[:, 1:2] - e0[:, 1:2] * e1[:, 0:1]
        nn = jnp.sqrt(n0 * n0 + n1 * n1 + n2 * n2 + 1e-12)
        inv_n = 1.0 / (nn + 1e-8)
        area = 0.5 * nn

        def ln(e):
            return jnp.sqrt(jnp.sum(e * e, axis=1, keepdims=True) + 1e-12)

        feats = jnp.concatenate(
            [n0 * inv_n, n1 * inv_n, n2 * inv_n, area, ln(e0), ln(e1), ln(e2),
             jnp.zeros((BN, 121), jnp.float32)], axis=1)
        o_ref[...] = feats

    return pl.pallas_call(
        body,
        grid=(FP // BN,),
        in_specs=[pl.BlockSpec((BN, 128), _blk)],
        out_specs=pl.BlockSpec((BN, 128), _blk),
        out_shape=jax.ShapeDtypeStruct((FP, 128), jnp.float32),
    )


def _tc_final():
    def body(s1, s2, s3, inv, w1, w2, w3, bp, pos, o_ref):
        iv = inv[...]
        y = jnp.dot(s1[...] * iv, w1[...], preferred_element_type=jnp.float32)
        y += jnp.dot(s2[...] * iv, w2[...], preferred_element_type=jnp.float32)
        y += jnp.dot(s3[...] * iv, w3[...], preferred_element_type=jnp.float32)
        o_ref[...] = pos[...] + y[:, :3] + bp[...][:, :3]

    return pl.pallas_call(
        body,
        grid=(NP // BN,),
        in_specs=[pl.BlockSpec((BN, 64), _blk),
                  pl.BlockSpec((BN, 64), _blk),
                  pl.BlockSpec((BN, 64), _blk),
                  pl.BlockSpec((BN, 64), _blk),
                  pl.BlockSpec((64, 128), lambda i: (0, 0)),
                  pl.BlockSpec((64, 128), lambda i: (0, 0)),
                  pl.BlockSpec((64, 128), lambda i: (0, 0)),
                  pl.BlockSpec((1, 128), lambda i: (0, 0)),
                  pl.BlockSpec((BN, 3), _blk)],
        out_specs=pl.BlockSpec((BN, 3), _blk),
        out_shape=jax.ShapeDtypeStruct((NP, 3), jnp.float32),
    )


_scatter64 = None
_scatter16 = None
_counts = None
_gather = None
_featgather = None


def _kernels():
    global _scatter64, _scatter16, _counts, _gather, _featgather
    if _scatter64 is None:
        _scatter64 = _make_scatter(4, True)
        _scatter16 = _make_scatter(1, True)
        _counts = _make_scatter(1, False)
        _gather = _make_gather()
        _featgather = _make_featgather()
    return _scatter64, _scatter16, _counts, _gather, _featgather


def _padw(w):
    k, m = w.shape
    return jnp.zeros((128, 128), jnp.float32).at[:k, :m].set(w)


def _row(b):
    return jnp.zeros((1, 128), jnp.float32).at[0, :b.shape[0]].set(b)


def kernel(pos, faces, W1, b1, W1a, b1a, Wp1, bp1, Wf1, bf1, W2, b2, W2a, b2a,
           Wp2, bp2, Wf2, bf2, W3, b3, W3a, b3a, Wp3, bp3):
    sc64, sc16, counts_k, gather_k, featgather_k = _kernels()
    f32 = jnp.float32

    pad_n = FP - F
    trash = (100352 + (jnp.arange(pad_n, dtype=jnp.int32) % 2048))
    f0 = jnp.concatenate([faces[:, 0], trash])
    f1 = jnp.concatenate([faces[:, 1], trash])
    f2 = jnp.concatenate([faces[:, 2], trash])
    pos_pad = jnp.zeros((NP, 128), f32).at[:N, :3].set(pos)

    Wt1, Wb1 = _padw(W1[:7]), _padw(W1[7:] / 3.0)
    Wt1a, Wb1a = _padw(W1a[:64]), _padw(W1a[64:] / 3.0)
    Wt2, Wb2 = _padw(W2[:64]), _padw(W2[64:] / 3.0)
    Wt2a, Wb2a = _padw(W2a[:64]), _padw(W2a[64:] / 3.0)
    Wt3, Wb3 = _padw(W3[:64]), _padw(W3[64:] / 3.0)
    Wt3a, Wb3a = _padw(W3a[:64]), _padw(W3a[64:] / 3.0)
    Wfp1, Wfp2 = _padw(Wf1 / 3.0), _padw(Wf2 / 3.0)
    Wpp1, Wpp2, Wpp3 = _padw(Wp1), _padw(Wp2), _padw(Wp3)
    zrow = jnp.zeros((1, 128), f32)
    beff1, beff1a = _row(b1), _row(b1a)
    beff2 = _row(b2 + bf1 @ W2[:64])
    nbias2 = _row(bf1 @ (W2[64:] / 3.0))
    beff2a = _row(b2a)
    beff3 = _row(b3 + bf2 @ W3[:64])
    nbias3 = _row(bf2 @ (W3[64:] / 3.0))
    beff3a = _row(b3a)
    bpsum = _row(bp1 + bp2 + bp3)

    inv_k, nodemm, facemm, geom, final_k = (
        _tc_inv(), _tc_nodemm(), _tc_facemm(), _tc_geom(), _tc_final())

    def xla_scatter(x):
        s = jnp.zeros((NP, 64), f32)
        h = x[:, :64]
        return s.at[f0].add(h).at[f1].add(h).at[f2].add(h)

    cnt = jnp.zeros((NP,), f32).at[f0].add(1.0).at[f1].add(1.0).at[f2].add(1.0)
    inv, ind = inv_k(cnt[:, None])

    vpack = featgather_k(pos_pad, f0, f1, f2)
    x = geom(vpack)

    def face_conv(x, Wt, Wb, beff, nbias, wide):
        spl = xla_scatter(x)
        y = nodemm(spl, inv, ind, Wb, nbias)
        g = gather_k(y, f0, f1, f2)
        return facemm(x, g, Wt, beff)

    x = face_conv(x, Wt1, Wb1[:64], beff1, zrow, False)
    x = face_conv(x, Wt1a, Wb1a[:64], beff1a, zrow, True)
    spl1 = xla_scatter(x)
    u = nodemm(spl1, inv, ind, Wfp1[:64], zrow)
    x = gather_k(u, f0, f1, f2)
    x = face_conv(x, Wt2, Wb2[:64], beff2, nbias2, True)
    x = face_conv(x, Wt2a, Wb2a[:64], beff2a, zrow, True)
    spl2 = xla_scatter(x)
    u = nodemm(spl2, inv, ind, Wfp2[:64], zrow)
    x = gather_k(u, f0, f1, f2)
    x = face_conv(x, Wt3, Wb3[:64], beff3, nbias3, True)
    x = face_conv(x, Wt3a, Wb3a[:64], beff3a, zrow, True)
    spl3 = xla_scatter(x)

    pos_out = final_k(spl1, spl2, spl3, inv,
                      Wpp1[:64], Wpp2[:64], Wpp3[:64], bpsum,
                      jnp.zeros((NP, 3), f32).at[:N].set(pos))
    return pos_out[:N]
